# Initial kernel scaffold; baseline (speedup 1.0000x reference)
import jax
import jax.numpy as jnp
from jax.experimental import pallas as pl


def kernel(x, emb1, emb2, W1, b1, W2, b2, Wl, bl):
    raise NotImplementedError("write your pallas kernel here")


# speedup vs baseline: 5.8396x; 5.8396x over previous
import functools

import jax
import jax.numpy as jnp
from jax import lax
from jax.experimental import pallas as pl
from jax.experimental.pallas import tpu as pltpu
from jax.experimental.pallas import tpu_sc as plsc

NREAL = 10000
NP = 10240
SEQ = 128
KTOP = 16
ALPHA = 0.1
NEG = -1.0e30
NSUB = 16
HALF = NP // 2
DUMMY = 128
HROWS = HALF + DUMMY
STRIP = NP // NSUB
NB = STRIP // 16
RED = HALF // NSUB
TILES = HALF // 128

@functools.cache
def _mesh():
    return plsc.VectorSubcoreMesh(core_axis_name="c", subcore_axis_name="s")


def _nv_body(e1, w1, b1, e2, w2, b2, nv1, nv2):
    dn = (((1,), (1,)), ((), ()))
    nv1[...] = jnp.tanh(ALPHA * (
        lax.dot_general(e1[...], w1[...], dn, preferred_element_type=jnp.float32)
        + b1[...]))
    nv2[...] = jnp.tanh(ALPHA * (
        lax.dot_general(e2[...], w2[...], dn, preferred_element_type=jnp.float32)
        + b2[...]))


def _nv_call(e1, w1, b1, e2, w2, b2):
    blk = lambda i: (i, 0)
    fix = lambda i: (0, 0)
    return pl.pallas_call(
        _nv_body,
        grid=(NP // 128,),
        in_specs=[
            pl.BlockSpec((128, SEQ), blk), pl.BlockSpec((SEQ, SEQ), fix),
            pl.BlockSpec((1, SEQ), fix),
            pl.BlockSpec((128, SEQ), blk), pl.BlockSpec((SEQ, SEQ), fix),
            pl.BlockSpec((1, SEQ), fix),
        ],
        out_specs=[pl.BlockSpec((128, SEQ), blk)] * 2,
        out_shape=[jax.ShapeDtypeStruct((NP, SEQ), jnp.float32)] * 2,
    )(e1, w1, b1, e2, w2, b2)


def _topk_body(nv1f, nv2f, nv1b, nv2b, colsT, wT):
    i = pl.program_id(0)
    dn = (((1,), (1,)), ((), ()))
    a = lax.dot_general(nv1b[...], nv2f[...], dn, preferred_element_type=jnp.float32)
    a = a - lax.dot_general(nv2b[...], nv1f[...], dn, preferred_element_type=jnp.float32)
    iota_c = lax.broadcasted_iota(jnp.int32, (128, NP), 1)
    a = jnp.where(iota_c < NREAL, a, NEG)
    lane = lax.broadcasted_iota(jnp.int32, (1, 128), 1)
    rowmask = (i * 128 + lane[0]) < NREAL
    for k in range(KTOP):
        m = jnp.max(a, axis=1)
        col = jnp.min(jnp.where(a == m[:, None], iota_c, jnp.int32(2 ** 30)),
                      axis=1)
        a = jnp.where(iota_c == col[:, None], NEG, a)
        w = jnp.maximum(jnp.tanh(ALPHA * m), 0.0)
        colsT[k, :] = col
        wT[k, :] = jnp.where(rowmask, w, 0.0)


def _topk_call(nv1, nv2):
    full = lambda i: (0, 0)
    blk = lambda i: (i, 0)
    out = lambda i: (0, i)
    return pl.pallas_call(
        _topk_body,
        grid=(NP // 128,),
        in_specs=[
            pl.BlockSpec((NP, SEQ), full), pl.BlockSpec((NP, SEQ), full),
            pl.BlockSpec((128, SEQ), blk), pl.BlockSpec((128, SEQ), blk),
        ],
        out_specs=[pl.BlockSpec((KTOP, 128), out)] * 2,
        out_shape=[jax.ShapeDtypeStruct((KTOP, NP), jnp.int32),
                   jax.ShapeDtypeStruct((KTOP, NP), jnp.float32)],
    )(nv1, nv2, nv1, nv2)


def _rsqrt_newton(x):
    ii = lax.bitcast_convert_type(x, jnp.int32)
    y = lax.bitcast_convert_type(jnp.int32(0x5F3759DF) - (ii >> 1), jnp.float32)
    for _ in range(3):
        y = y * (1.5 - 0.5 * x * y * y)
    return y


def _bcast(v, j):
    idx = jnp.reshape(jnp.zeros((16,), jnp.int32) + j, (16, 1))
    return lax.gather(
        v, idx,
        lax.GatherDimensionNumbers(offset_dims=(), collapsed_slice_dims=(0,),
                                   start_index_map=(0,)),
        (1,), mode=lax.GatherScatterMode.PROMISE_IN_BOUNDS)


@functools.cache
def _build_deg_kernel():
  @functools.partial(
      pl.kernel,
      out_type=jax.ShapeDtypeStruct((NP,), jnp.float32),
      mesh=_mesh(),
      compiler_params=pltpu.CompilerParams(needs_layout_passes=False),
      scratch_types=[
          pltpu.VMEM((KTOP, STRIP), jnp.int32),
          pltpu.VMEM((KTOP, STRIP), jnp.float32),
          pltpu.VMEM((NP,), jnp.float32),
          pltpu.VMEM((NSUB * 128,), jnp.float32),
          pltpu.VMEM((128,), jnp.float32),
          pltpu.VMEM_SHARED((NSUB * NP,), jnp.float32),
      ])
  def _deg_kernel(colsT, wT, dinv_out, colsv, wv, hist, redbuf, chunk, shared):
    c = lax.axis_index("c")
    s = lax.axis_index("s")

    def zero(i, _):
        hist[pl.ds(pl.multiple_of(i * 16, 16), 16)] = jnp.zeros((16,),
                                                                jnp.float32)
        return 0
    lax.fori_loop(0, NP // 16, zero, 0)

    soff = pl.multiple_of(s * STRIP, 128)
    pltpu.sync_copy(colsT.at[:, pl.ds(soff, STRIP)], colsv)
    pltpu.sync_copy(wT.at[:, pl.ds(soff, STRIP)], wv)

    for k in range(KTOP):
        def accum(j, _, k=k):
            jo = pl.multiple_of(j * 16, 16)
            plsc.addupdate_scatter(hist, [colsv[k, pl.ds(jo, 16)]],
                                   wv[k, pl.ds(jo, 16)])
            return 0
        lax.fori_loop(0, NB, accum, 0)

    pltpu.sync_copy(hist, shared.at[pl.ds(pl.multiple_of(s * NP, 128), NP)])
    plsc.subcore_barrier()

    for t_i in range(3):
        t = s + NSUB * t_i

        @pl.when(t < TILES)
        def _():
            g0 = pl.multiple_of(c * HALF + t * 128, 128)
            for tt in range(NSUB):
                pltpu.sync_copy(
                    shared.at[pl.ds(pl.multiple_of(tt * NP + g0, 128), 128)],
                    redbuf.at[pl.ds(tt * 128, 128)])
            for cc in range(8):
                acc = jnp.ones((16,), jnp.float32)
                for tt in range(NSUB):
                    acc = acc + redbuf[pl.ds(tt * 128 + cc * 16, 16)]
                chunk[pl.ds(cc * 16, 16)] = _rsqrt_newton(acc)
            pltpu.sync_copy(chunk, dinv_out.at[pl.ds(g0, 128)])

  return _deg_kernel


def _deg_call(colsT, wT):
    return _build_deg_kernel()(colsT, wT)


@functools.cache
def _build_hop_kernel():
  @functools.partial(
      pl.kernel,
      out_type=jax.ShapeDtypeStruct((NP, SEQ), jnp.float32),
      mesh=_mesh(),
      compiler_params=pltpu.CompilerParams(needs_layout_passes=False),
      scratch_types=[
          pltpu.VMEM((KTOP, STRIP), jnp.int32),
          pltpu.VMEM((KTOP, STRIP), jnp.float32),
          pltpu.VMEM((NP,), jnp.float32),
          pltpu.VMEM((16, SEQ), jnp.float32),
          pltpu.VMEM((17 * 16, SEQ), jnp.float32),
          pltpu.VMEM((128, SEQ), jnp.float32),
          pltpu.VMEM((128,), jnp.int32),
          pltpu.VMEM((128,), jnp.int32),
          pltpu.VMEM((16,), jnp.int32),
          pltpu.VMEM((17, 16), jnp.float32),
          pltpu.VMEM_SHARED((HROWS, SEQ), jnp.float32),
      ])
  def _hop_kernel(h, colsT, wT, dinv, hout,
                  colsv, wv, dinvv, hbuf, msg, zbuf, idxA, idxB, idxC, normb,
                  hnew):
    c = lax.axis_index("c")
    s = lax.axis_index("s")
    base = c * HALF
    iota16 = lax.iota(jnp.int32, 16)

    def zloop(i, _):
        for cc in range(SEQ // 16):
            zbuf[i, pl.ds(cc * 16, 16)] = jnp.zeros((16,), jnp.float32)
        return 0
    lax.fori_loop(0, 128, zloop, 0)
    for t_i in range(3):
        t = s + NSUB * t_i

        @pl.when(t < HROWS // 128)
        def _():
            pltpu.sync_copy(
                zbuf, hnew.at[pl.ds(pl.multiple_of(t * 128, 128), 128)])
    plsc.subcore_barrier()

    soff = pl.multiple_of(s * STRIP, 128)
    pltpu.sync_copy(colsT.at[:, pl.ds(soff, STRIP)], colsv)
    pltpu.sync_copy(wT.at[:, pl.ds(soff, STRIP)], wv)
    pltpu.sync_copy(dinv, dinvv)

    def batch(b, _):
        bo = pl.multiple_of(b * 16, 16)
        r0 = pl.multiple_of(s * STRIP + bo, 16)
        pltpu.sync_copy(h.at[pl.ds(r0, 16)], hbuf)
        dsrc = dinvv[pl.ds(r0, 16)]

        for k in range(KTOP + 1):
            if k < KTOP:
                dstv = colsv[k, pl.ds(bo, 16)]
                wvk = wv[k, pl.ds(bo, 16)]
            else:
                dstv = r0 + iota16
                wvk = jnp.ones((16,), jnp.float32)
            ddst = plsc.load_gather(dinvv, [dstv])
            normb[k, :] = dsrc * wvk * ddst
            route = dstv - base
            ok = (route >= 0) & (route < HALF)
            ridx = jnp.where(ok, route,
                             HALF + ((dstv + iota16) & (DUMMY - 1)))
            if k < 8:
                idxA[pl.ds(k * 16, 16)] = ridx
            elif k < KTOP:
                idxB[pl.ds((k - 8) * 16, 16)] = ridx
            else:
                idxC[...] = ridx

        def p2(j, _):
            hrow = [hbuf[j, pl.ds(cc * 16, 16)] for cc in range(SEQ // 16)]
            for k in range(KTOP + 1):
                nb = _bcast(normb[k, :], j)
                row = k * 16 + j
                for cc in range(SEQ // 16):
                    msg[row, pl.ds(cc * 16, 16)] = nb * hrow[cc]
            return 0
        lax.fori_loop(0, 16, p2, 0)

        pltpu.sync_copy(msg.at[pl.ds(0, 128)], hnew.at[idxA], add=True)
        pltpu.sync_copy(msg.at[pl.ds(128, 128)], hnew.at[idxB], add=True)
        pltpu.sync_copy(msg.at[pl.ds(256, 16)], hnew.at[idxC], add=True)
        return 0
    lax.fori_loop(0, NB, batch, 0)
    plsc.subcore_barrier()

    ob = pl.multiple_of(s * RED, 64)
    pltpu.sync_copy(hnew.at[pl.ds(ob, RED)],
                    hout.at[pl.ds(pl.multiple_of(base + ob, 64), RED)])

  return _hop_kernel


def _hop_call(h, colsT, wT, dinv):
    return _build_hop_kernel()(h, colsT, wT, dinv)


def _lin_body(hb, wl, bl, ob):
    dn = (((1,), (1,)), ((), ()))
    ob[...] = jnp.maximum(
        lax.dot_general(hb[...], wl[...], dn, preferred_element_type=jnp.float32)
        + bl[...], 0.0)


def _lin_call(h, wl, bl):
    blk = lambda i: (i, 0)
    fix = lambda i: (0, 0)
    return pl.pallas_call(
        _lin_body,
        grid=(NP // 128,),
        in_specs=[pl.BlockSpec((128, SEQ), blk), pl.BlockSpec((SEQ, SEQ), fix),
                  pl.BlockSpec((1, SEQ), fix)],
        out_specs=pl.BlockSpec((128, SEQ), blk),
        out_shape=jax.ShapeDtypeStruct((NP, SEQ), jnp.float32),
    )(h, wl, bl)


def kernel(x, emb1, emb2, W1, b1, W2, b2, Wl, bl):
    pad = ((0, NP - NREAL), (0, 0))
    e1 = jnp.pad(emb1, pad)
    e2 = jnp.pad(emb2, pad)
    xp = jnp.pad(x, pad)
    nv1, nv2 = _nv_call(e1, W1, b1.reshape(1, -1), e2, W2, b2.reshape(1, -1))
    colsT, wT = _topk_call(nv1, nv2)
    dinv = _deg_call(colsT, wT)
    h = xp
    for _ in range(KTOP):
        h = _hop_call(h, colsT, wT, dinv)
    out = _lin_call(h, Wl, bl.reshape(1, -1))
    return out[:NREAL]
